# baseline (speedup 1.0000x reference)
import functools

import jax
import jax.numpy as jnp
from jax import lax
from jax.experimental import pallas as pl
from jax.experimental.pallas import tpu as pltpu
from jax.experimental.pallas import tpu_sc as plsc

_NC = 2
_NS = 16
_NW = _NC * _NS
_LW = 128


def _gather_mean(emb, idx3, ctx, rows_per_w, chunks):
  v, d = emb.shape
  b = _NW * rows_per_w
  per_w = chunks * _LW
  mesh = plsc.VectorSubcoreMesh(core_axis_name="c", subcore_axis_name="s")

  @functools.partial(
      pl.kernel,
      mesh=mesh,
      compiler_params=pltpu.CompilerParams(use_tc_tiling_on_sc=False),
      out_type=jax.ShapeDtypeStruct((b, d), jnp.float32),
      scratch_types=[
          pltpu.VMEM((chunks, _LW), jnp.int32),
          pltpu.VMEM((per_w, d), jnp.float32),
          pltpu.VMEM((rows_per_w, d), jnp.float32),
          pltpu.SemaphoreType.DMA,
      ],
  )
  def body(emb_hbm, idx_hbm, out_hbm, idx_v, rows_v, hid_v, sem):
    wid = lax.axis_index("s") * _NC + lax.axis_index("c")
    pltpu.sync_copy(idx_hbm.at[wid], idx_v)
    for j in range(chunks):
      pltpu.async_copy(emb_hbm.at[idx_v.at[j]],
                       rows_v.at[pl.ds(j * _LW, _LW)], sem)
    for j in range(chunks):
      pltpu.make_async_copy(emb_hbm.at[idx_v.at[j]],
                            rows_v.at[pl.ds(j * _LW, _LW)], sem).wait()
    inv = jnp.float32(1.0 / ctx)

    def row_body(r, carry):
      base = r * ctx
      acc = rows_v[base, :]
      for j in range(1, ctx):
        acc = acc + rows_v[base + j, :]
      hid_v[r, :] = acc * inv
      return carry

    lax.fori_loop(0, rows_per_w, row_body, 0)
    pltpu.sync_copy(hid_v, out_hbm.at[pl.ds(wid * rows_per_w, rows_per_w)])

  return body(emb, idx3)


def _transpose_body(w_ref, wt_ref):
  wt_ref[...] = w_ref[...].T.astype(jnp.bfloat16)


def _mm_logsoftmax_body(v, h_ref, wt_ref, o_ref):
  vp = wt_ref.shape[1]
  logits = lax.dot_general(h_ref[...], wt_ref[...],
                           (((1,), (0,)), ((), ())),
                           preferred_element_type=jnp.float32)
  head = logits[:, : vp - 128]
  tail = logits[:, vp - 128:]
  col = jax.lax.broadcasted_iota(jnp.int32, tail.shape, 1)
  tail_m = jnp.where(col < 128 - (vp - v), tail, -jnp.inf)
  m = jnp.maximum(jnp.max(head, axis=1, keepdims=True),
                  jnp.max(tail_m, axis=1, keepdims=True))
  s = (jnp.sum(jnp.exp(head - m), axis=1, keepdims=True)
       + jnp.sum(jnp.exp(tail_m - m), axis=1, keepdims=True))
  o_ref[...] = (logits - (m + jnp.log(s)))[:, :v]


def kernel(inputs, emb, W_out):
  b, ctx = inputs.shape
  v, d = emb.shape
  total = b * ctx
  per_w = total // _NW
  chunks = per_w // _LW
  rows_per_w = b // _NW

  idx3 = inputs.astype(jnp.int32).reshape(_NW, chunks, _LW)
  hidden = _gather_mean(emb, idx3, ctx, rows_per_w, chunks)

  vp = ((v + 127) // 128) * 128
  t_blk = 2048
  wt = pl.pallas_call(
      _transpose_body,
      grid=(pl.cdiv(v, t_blk),),
      in_specs=[pl.BlockSpec((t_blk, d), lambda i: (i, 0))],
      out_specs=pl.BlockSpec((d, t_blk), lambda i: (0, i)),
      out_shape=jax.ShapeDtypeStruct((d, vp), jnp.bfloat16),
  )(W_out)

  b_blk = 16
  out = pl.pallas_call(
      functools.partial(_mm_logsoftmax_body, v),
      grid=(b // b_blk,),
      in_specs=[
          pl.BlockSpec((b_blk, d), lambda i: (i, 0)),
          pl.BlockSpec((d, vp), lambda i: (0, 0)),
      ],
      out_specs=pl.BlockSpec((b_blk, v), lambda i: (i, 0)),
      out_shape=jax.ShapeDtypeStruct((b, v), jnp.float32),
  )(hidden.astype(jnp.bfloat16), wt)
  return out


# speedup vs baseline: 1.7931x; 1.7931x over previous
import functools

import jax
import jax.numpy as jnp
from jax import lax
from jax.experimental import pallas as pl
from jax.experimental.pallas import tpu as pltpu
from jax.experimental.pallas import tpu_sc as plsc

_NC = 2
_NS = 16
_NW = _NC * _NS
_LW = 128


def _gather_mean(emb, idx3, ctx, rows_per_w, chunks):
  v, d = emb.shape
  b = _NW * rows_per_w
  per_w = chunks * _LW
  mesh = plsc.VectorSubcoreMesh(core_axis_name="c", subcore_axis_name="s")

  @functools.partial(
      pl.kernel,
      mesh=mesh,
      compiler_params=pltpu.CompilerParams(use_tc_tiling_on_sc=False),
      out_type=jax.ShapeDtypeStruct((b, d), jnp.float32),
      scratch_types=[
          pltpu.VMEM((chunks, _LW), jnp.int32),
          pltpu.VMEM((per_w, d), jnp.float32),
          pltpu.VMEM((rows_per_w, d), jnp.float32),
          pltpu.SemaphoreType.DMA,
      ],
  )
  def body(emb_hbm, idx_hbm, out_hbm, idx_v, rows_v, hid_v, sem):
    wid = lax.axis_index("s") * _NC + lax.axis_index("c")
    pltpu.sync_copy(idx_hbm.at[wid], idx_v)
    for j in range(chunks):
      pltpu.async_copy(emb_hbm.at[idx_v.at[j]],
                       rows_v.at[pl.ds(j * _LW, _LW)], sem)
    for j in range(chunks):
      pltpu.make_async_copy(emb_hbm.at[idx_v.at[j]],
                            rows_v.at[pl.ds(j * _LW, _LW)], sem).wait()
    inv = jnp.float32(1.0 / ctx)

    def row_body(r, carry):
      base = r * ctx
      acc = rows_v[base, :]
      for j in range(1, ctx):
        acc = acc + rows_v[base + j, :]
      hid_v[r, :] = acc * inv
      return carry

    lax.fori_loop(0, rows_per_w, row_body, 0)
    pltpu.sync_copy(hid_v, out_hbm.at[pl.ds(wid * rows_per_w, rows_per_w)])

  return body(emb, idx3)


def _stats_body(nv, w_ref, h_ref, lse_ref, m_sc, s_sc):
  j = pl.program_id(0)

  @pl.when(j == 0)
  def _init():
    m_sc[...] = jnp.full(m_sc.shape, -jnp.inf, m_sc.dtype)
    s_sc[...] = jnp.zeros(s_sc.shape, s_sc.dtype)

  w = w_ref[...].astype(jnp.bfloat16)
  lg = lax.dot_general(w, h_ref[...], (((1,), (0,)), ((), ())),
                       preferred_element_type=jnp.float32)
  mp = jnp.max(lg, axis=0, keepdims=True)
  m_old = m_sc[0:1, :]
  m_new = jnp.maximum(m_old, mp)
  s_new = (s_sc[0:1, :] * jnp.exp(m_old - m_new)
           + jnp.sum(jnp.exp(lg - m_new), axis=0, keepdims=True))
  m_sc[0:1, :] = m_new
  s_sc[0:1, :] = s_new

  @pl.when(j == nv - 1)
  def _fin():
    lse_ref[...] = jnp.broadcast_to(m_new + jnp.log(s_new), lse_ref.shape)


def _write_body(w_ref, h_ref, lse_ref, o_ref):
  w = w_ref[...].astype(jnp.bfloat16)
  lg = lax.dot_general(w, h_ref[...], (((1,), (0,)), ((), ())),
                       preferred_element_type=jnp.float32)
  o_ref[...] = lg - lse_ref[0:1, :]


def kernel(inputs, emb, W_out):
  b, ctx = inputs.shape
  v, d = emb.shape
  total = b * ctx
  per_w = total // _NW
  chunks = per_w // _LW
  rows_per_w = b // _NW

  idx3 = inputs.astype(jnp.int32).reshape(_NW, chunks, _LW)
  hidden = _gather_mean(emb, idx3, ctx, rows_per_w, chunks)
  h_t = hidden.T.astype(jnp.bfloat16)

  v_blk = 2000
  nv = v // v_blk
  lse = pl.pallas_call(
      functools.partial(_stats_body, nv),
      grid=(nv,),
      in_specs=[
          pl.BlockSpec((v_blk, d), lambda j: (j, 0)),
          pl.BlockSpec((d, b), lambda j: (0, 0)),
      ],
      out_specs=pl.BlockSpec((8, b), lambda j: (0, 0)),
      out_shape=jax.ShapeDtypeStruct((8, b), jnp.float32),
      scratch_shapes=[
          pltpu.VMEM((8, b), jnp.float32),
          pltpu.VMEM((8, b), jnp.float32),
      ],
  )(W_out, h_t)

  out_t = pl.pallas_call(
      _write_body,
      grid=(nv,),
      in_specs=[
          pl.BlockSpec((v_blk, d), lambda j: (j, 0)),
          pl.BlockSpec((d, b), lambda j: (0, 0)),
          pl.BlockSpec((8, b), lambda j: (0, 0)),
      ],
      out_specs=pl.BlockSpec((v_blk, b), lambda j: (j, 0)),
      out_shape=jax.ShapeDtypeStruct((v, b), jnp.float32),
  )(W_out, h_t, lse)
  return out_t.T


# speedup vs baseline: 1.8535x; 1.0337x over previous
import functools

import jax
import jax.numpy as jnp
from jax import lax
from jax.experimental import pallas as pl
from jax.experimental.pallas import tpu as pltpu
from jax.experimental.pallas import tpu_sc as plsc

_NC = 2
_NS = 16
_NW = _NC * _NS
_LW = 128


def _gather_mean(emb, idx3, ctx, rows_per_w, chunks):
  v, d = emb.shape
  b = _NW * rows_per_w
  per_w = chunks * _LW
  mesh = plsc.VectorSubcoreMesh(core_axis_name="c", subcore_axis_name="s")

  @functools.partial(
      pl.kernel,
      mesh=mesh,
      compiler_params=pltpu.CompilerParams(use_tc_tiling_on_sc=False),
      out_type=jax.ShapeDtypeStruct((b, d), jnp.float32),
      scratch_types=[
          pltpu.VMEM((chunks, _LW), jnp.int32),
          pltpu.VMEM((per_w, d), jnp.float32),
          pltpu.VMEM((rows_per_w, d), jnp.float32),
          pltpu.SemaphoreType.DMA,
      ],
  )
  def body(emb_hbm, idx_hbm, out_hbm, idx_v, rows_v, hid_v, sem):
    wid = lax.axis_index("s") * _NC + lax.axis_index("c")
    pltpu.sync_copy(idx_hbm.at[wid], idx_v)
    for j in range(chunks):
      pltpu.async_copy(emb_hbm.at[idx_v.at[j]],
                       rows_v.at[pl.ds(j * _LW, _LW)], sem)
    for j in range(chunks):
      pltpu.make_async_copy(emb_hbm.at[idx_v.at[j]],
                            rows_v.at[pl.ds(j * _LW, _LW)], sem).wait()
    inv = jnp.float32(1.0 / ctx)

    def row_body(r, carry):
      base = r * ctx
      acc = rows_v[base, :]
      for j in range(1, ctx):
        acc = acc + rows_v[base + j, :]
      hid_v[r, :] = acc * inv
      return carry

    lax.fori_loop(0, rows_per_w, row_body, 0)
    pltpu.sync_copy(hid_v, out_hbm.at[pl.ds(wid * rows_per_w, rows_per_w)])

  return body(emb, idx3)


def _stats_body(h_ref, wt_ref, lse_ref):
  lg = lax.dot_general(h_ref[...], wt_ref[...], (((1,), (0,)), ((), ())),
                       preferred_element_type=jnp.float32)
  m = jnp.max(lg, axis=1, keepdims=True)
  s = jnp.sum(jnp.exp(lg - m), axis=1, keepdims=True)
  lse_ref[...] = m + jnp.log(s)


def _write_body(wt_ref, h_ref, lse_ref, o_ref):
  lg = lax.dot_general(wt_ref[...], h_ref[...], (((0,), (0,)), ((), ())),
                       preferred_element_type=jnp.float32)
  o_ref[...] = lg - lse_ref[0:1, :]


def kernel(inputs, emb, W_out):
  b, ctx = inputs.shape
  v, d = emb.shape
  total = b * ctx
  per_w = total // _NW
  chunks = per_w // _LW
  rows_per_w = b // _NW

  idx3 = inputs.astype(jnp.int32).reshape(_NW, chunks, _LW)
  hidden = _gather_mean(emb, idx3, ctx, rows_per_w, chunks)
  h_bf = hidden.astype(jnp.bfloat16)
  h_t = h_bf.T
  wt = W_out.T.astype(jnp.bfloat16)

  b_blk = 16
  lse = pl.pallas_call(
      _stats_body,
      grid=(b // b_blk,),
      in_specs=[
          pl.BlockSpec((b_blk, d), lambda i: (i, 0)),
          pl.BlockSpec((d, v), lambda i: (0, 0)),
      ],
      out_specs=pl.BlockSpec((b_blk, 1), lambda i: (i, 0)),
      out_shape=jax.ShapeDtypeStruct((b, 1), jnp.float32),
  )(h_bf, wt)
  lse_row = jnp.broadcast_to(lse.reshape(1, b), (8, b))

  v_blk = 2048
  out_t = pl.pallas_call(
      _write_body,
      grid=(pl.cdiv(v, v_blk),),
      in_specs=[
          pl.BlockSpec((d, v_blk), lambda j: (0, j)),
          pl.BlockSpec((d, b), lambda j: (0, 0)),
          pl.BlockSpec((8, b), lambda j: (0, 0)),
      ],
      out_specs=pl.BlockSpec((v_blk, b), lambda j: (j, 0)),
      out_shape=jax.ShapeDtypeStruct((v, b), jnp.float32),
  )(wt, h_t, lse_row)
  return out_t.T


# speedup vs baseline: 1.9320x; 1.0423x over previous
import functools

import jax
import jax.numpy as jnp
from jax import lax
from jax.experimental import pallas as pl
from jax.experimental.pallas import tpu as pltpu
from jax.experimental.pallas import tpu_sc as plsc

_NC = 2
_NS = 16
_NW = _NC * _NS
_LW = 128


def _gather_mean(emb, idx3, ctx, rows_per_w, chunks):
  v, d = emb.shape
  b = _NW * rows_per_w
  per_w = chunks * _LW
  mesh = plsc.VectorSubcoreMesh(core_axis_name="c", subcore_axis_name="s")

  @functools.partial(
      pl.kernel,
      mesh=mesh,
      compiler_params=pltpu.CompilerParams(use_tc_tiling_on_sc=False),
      out_type=jax.ShapeDtypeStruct((b, d), jnp.float32),
      scratch_types=[
          pltpu.VMEM((chunks, _LW), jnp.int32),
          pltpu.VMEM((per_w, d), jnp.float32),
          pltpu.VMEM((rows_per_w, d), jnp.float32),
          pltpu.SemaphoreType.DMA,
      ],
  )
  def body(emb_hbm, idx_hbm, out_hbm, idx_v, rows_v, hid_v, sem):
    wid = lax.axis_index("s") * _NC + lax.axis_index("c")
    pltpu.sync_copy(idx_hbm.at[wid], idx_v)
    for j in range(chunks):
      pltpu.async_copy(emb_hbm.at[idx_v.at[j]],
                       rows_v.at[pl.ds(j * _LW, _LW)], sem)
    for j in range(chunks):
      pltpu.make_async_copy(emb_hbm.at[idx_v.at[j]],
                            rows_v.at[pl.ds(j * _LW, _LW)], sem).wait()
    inv = jnp.float32(1.0 / ctx)

    def row_body(r, carry):
      base = r * ctx
      acc = rows_v[base, :]
      for j in range(1, ctx):
        acc = acc + rows_v[base + j, :]
      hid_v[r, :] = acc * inv
      return carry

    lax.fori_loop(0, rows_per_w, row_body, 0)
    pltpu.sync_copy(hid_v, out_hbm.at[pl.ds(wid * rows_per_w, rows_per_w)])

  return body(emb, idx3)


def _stats_body(nv, nvalid_last, wt_ref, h_ref, lse_ref, m_sc, s_sc):
  j = pl.program_id(0)

  @pl.when(j == 0)
  def _init():
    m_sc[...] = jnp.full(m_sc.shape, -jnp.inf, m_sc.dtype)
    s_sc[...] = jnp.zeros(s_sc.shape, s_sc.dtype)

  lg = lax.dot_general(wt_ref[...], h_ref[...], (((0,), (0,)), ((), ())),
                       preferred_element_type=jnp.float32)

  def upd(lgx):
    mp = jnp.max(lgx, axis=0, keepdims=True)
    m_old = m_sc[0:1, :]
    m_new = jnp.maximum(m_old, mp)
    s_new = (s_sc[0:1, :] * jnp.exp(m_old - m_new)
             + jnp.sum(jnp.exp(lgx - m_new), axis=0, keepdims=True))
    m_sc[0:1, :] = m_new
    s_sc[0:1, :] = s_new
    return m_new, s_new

  @pl.when(j < nv - 1)
  def _plain():
    upd(lg)

  @pl.when(j == nv - 1)
  def _last():
    row = lax.broadcasted_iota(jnp.int32, lg.shape, 0)
    m_new, s_new = upd(jnp.where(row < nvalid_last, lg, -jnp.inf))
    lse_ref[...] = jnp.broadcast_to(m_new + jnp.log(s_new), lse_ref.shape)


def _write_body(wt_ref, h_ref, lse_ref, o_ref):
  lg = lax.dot_general(wt_ref[...], h_ref[...], (((0,), (0,)), ((), ())),
                       preferred_element_type=jnp.float32)
  o_ref[...] = lg - lse_ref[0:1, :]


def kernel(inputs, emb, W_out):
  b, ctx = inputs.shape
  v, d = emb.shape
  total = b * ctx
  per_w = total // _NW
  chunks = per_w // _LW
  rows_per_w = b // _NW

  idx3 = inputs.astype(jnp.int32).reshape(_NW, chunks, _LW)
  hidden = _gather_mean(emb, idx3, ctx, rows_per_w, chunks)
  h_bf = hidden.astype(jnp.bfloat16)
  h_t = h_bf.T
  wt = W_out.T.astype(jnp.bfloat16)

  v_blk = 4096
  nv = pl.cdiv(v, v_blk)
  nvalid_last = v - (nv - 1) * v_blk
  lse_row = pl.pallas_call(
      functools.partial(_stats_body, nv, nvalid_last),
      grid=(nv,),
      in_specs=[
          pl.BlockSpec((d, v_blk), lambda j: (0, j)),
          pl.BlockSpec((d, b), lambda j: (0, 0)),
      ],
      out_specs=pl.BlockSpec((8, b), lambda j: (0, 0)),
      out_shape=jax.ShapeDtypeStruct((8, b), jnp.float32),
      scratch_shapes=[
          pltpu.VMEM((8, b), jnp.float32),
          pltpu.VMEM((8, b), jnp.float32),
      ],
  )(wt, h_t)

  out_t = pl.pallas_call(
      _write_body,
      grid=(pl.cdiv(v, v_blk),),
      in_specs=[
          pl.BlockSpec((d, v_blk), lambda j: (0, j)),
          pl.BlockSpec((d, b), lambda j: (0, 0)),
          pl.BlockSpec((8, b), lambda j: (0, 0)),
      ],
      out_specs=pl.BlockSpec((v_blk, b), lambda j: (j, 0)),
      out_shape=jax.ShapeDtypeStruct((v, b), jnp.float32),
  )(wt, h_t, lse_row)
  return out_t.T


# speedup vs baseline: 1.9430x; 1.0057x over previous
import functools

import jax
import jax.numpy as jnp
from jax import lax
from jax.experimental import pallas as pl
from jax.experimental.pallas import tpu as pltpu
from jax.experimental.pallas import tpu_sc as plsc

_NC = 2
_NS = 16
_NW = _NC * _NS
_LW = 128


def _gather_mean(emb, idx3, ctx, rows_per_w, chunks):
  v, d = emb.shape
  b = _NW * rows_per_w
  per_w = chunks * _LW
  mesh = plsc.VectorSubcoreMesh(core_axis_name="c", subcore_axis_name="s")

  @functools.partial(
      pl.kernel,
      mesh=mesh,
      compiler_params=pltpu.CompilerParams(use_tc_tiling_on_sc=False),
      out_type=jax.ShapeDtypeStruct((b, d), jnp.float32),
      scratch_types=[
          pltpu.VMEM((chunks, _LW), jnp.int32),
          pltpu.VMEM((per_w, d), jnp.float32),
          pltpu.VMEM((rows_per_w, d), jnp.float32),
          pltpu.SemaphoreType.DMA,
      ],
  )
  def body(emb_hbm, idx_hbm, out_hbm, idx_v, rows_v, hid_v, sem):
    wid = lax.axis_index("s") * _NC + lax.axis_index("c")
    pltpu.sync_copy(idx_hbm.at[wid], idx_v)
    for j in range(chunks):
      pltpu.async_copy(emb_hbm.at[idx_v.at[j]],
                       rows_v.at[pl.ds(j * _LW, _LW)], sem)
    for j in range(chunks):
      pltpu.make_async_copy(emb_hbm.at[idx_v.at[j]],
                            rows_v.at[pl.ds(j * _LW, _LW)], sem).wait()
    inv = jnp.float32(1.0 / ctx)

    def row_body(r, carry):
      base = r * ctx
      acc = rows_v[base, :]
      for j in range(1, ctx):
        acc = acc + rows_v[base + j, :]
      hid_v[r, :] = acc * inv
      return carry

    lax.fori_loop(0, rows_per_w, row_body, 0)
    pltpu.sync_copy(hid_v, out_hbm.at[pl.ds(wid * rows_per_w, rows_per_w)])

  return body(emb, idx3)


def _stats_body(nv, nvalid_last, wt_ref, h_ref, lse_ref, m_sc, s_sc):
  j = pl.program_id(0)

  @pl.when(j == 0)
  def _init():
    m_sc[...] = jnp.full(m_sc.shape, -jnp.inf, m_sc.dtype)
    s_sc[...] = jnp.zeros(s_sc.shape, s_sc.dtype)

  lg = lax.dot_general(wt_ref[...], h_ref[...], (((0,), (0,)), ((), ())),
                       preferred_element_type=jnp.float32)

  def upd(lgx):
    mp = jnp.max(lgx, axis=0, keepdims=True)
    m_old = m_sc[0:1, :]
    m_new = jnp.maximum(m_old, mp)
    s_new = (s_sc[0:1, :] * jnp.exp(m_old - m_new)
             + jnp.sum(jnp.exp(lgx - m_new), axis=0, keepdims=True))
    m_sc[0:1, :] = m_new
    s_sc[0:1, :] = s_new
    return m_new, s_new

  @pl.when(j < nv - 1)
  def _plain():
    upd(lg)

  @pl.when(j == nv - 1)
  def _last():
    row = lax.broadcasted_iota(jnp.int32, lg.shape, 0)
    m_new, s_new = upd(jnp.where(row < nvalid_last, lg, -jnp.inf))
    lse_ref[...] = jnp.broadcast_to(m_new + jnp.log(s_new), lse_ref.shape)


def _write_body(wt_ref, h_ref, lse_ref, o_ref):
  lg = lax.dot_general(wt_ref[...], h_ref[...], (((0,), (0,)), ((), ())),
                       preferred_element_type=jnp.float32)
  o_ref[...] = lg - lse_ref[0:1, :]


def kernel(inputs, emb, W_out):
  b, ctx = inputs.shape
  v, d = emb.shape
  total = b * ctx
  per_w = total // _NW
  chunks = per_w // _LW
  rows_per_w = b // _NW

  idx3 = inputs.astype(jnp.int32).reshape(_NW, chunks, _LW)
  hidden = _gather_mean(emb, idx3, ctx, rows_per_w, chunks)
  h_bf = hidden.astype(jnp.bfloat16)
  h_t = h_bf.T
  wt = W_out.T.astype(jnp.bfloat16)

  v_blk = 4096
  nv = pl.cdiv(v, v_blk)
  nvalid_last = v - (nv - 1) * v_blk
  lse_row = pl.pallas_call(
      functools.partial(_stats_body, nv, nvalid_last),
      grid=(nv,),
      in_specs=[
          pl.BlockSpec((d, v_blk), lambda j: (0, j)),
          pl.BlockSpec((d, b), lambda j: (0, 0)),
      ],
      out_specs=pl.BlockSpec((8, b), lambda j: (0, 0)),
      out_shape=jax.ShapeDtypeStruct((8, b), jnp.float32),
      scratch_shapes=[
          pltpu.VMEM((8, b), jnp.float32),
          pltpu.VMEM((8, b), jnp.float32),
      ],
  )(wt, h_t)

  w_blk = 2048
  out_t = pl.pallas_call(
      _write_body,
      grid=(pl.cdiv(v, w_blk),),
      in_specs=[
          pl.BlockSpec((d, w_blk), lambda j: (0, j)),
          pl.BlockSpec((d, b), lambda j: (0, 0)),
          pl.BlockSpec((8, b), lambda j: (0, 0)),
      ],
      out_specs=pl.BlockSpec((w_blk, b), lambda j: (j, 0)),
      out_shape=jax.ShapeDtypeStruct((v, b), jnp.float32),
  )(wt, h_t, lse_row)
  return out_t.T
